# baseline (speedup 1.0000x reference)
import jax
import jax.numpy as jnp
from jax.experimental import pallas as pl

_BLK = 4000


def _attn_block(feat_ref, w_ref, o_ref):
    a = feat_ref[...] @ w_ref[...]
    o_ref[...] = jnp.where(a >= 0.0, a, 0.3 * a)


def _attn_scores(feat, w):
    e, k = feat.shape
    return pl.pallas_call(
        _attn_block,
        grid=(e // _BLK,),
        in_specs=[
            pl.BlockSpec((_BLK, k), lambda i: (i, 0)),
            pl.BlockSpec((k, 1), lambda i: (0, 0)),
        ],
        out_specs=pl.BlockSpec((_BLK, 1), lambda i: (i, 0)),
        out_shape=jax.ShapeDtypeStruct((e, 1), feat.dtype),
    )(feat, w)[:, 0]


def _row_softmax(rows, vals, nrows):
    m = jax.ops.segment_max(vals, rows, num_segments=nrows)
    ev = jnp.exp(vals - m[rows])
    s = jax.ops.segment_sum(ev, rows, num_segments=nrows)
    return ev / s[rows]


def kernel(ent_emb, rel_emb, attr_emb, all_matix, attr_matrix, attn_kernels):
    node_size = ent_emb.shape[0]
    rel_size = rel_emb.shape[0]
    attr_size = attr_emb.shape[0]
    layers, nhead = attn_kernels.shape[0], attn_kernels.shape[1]

    ee = all_matix[:, 0:2].astype(jnp.int32)
    keys = ee[:, 0] * node_size + ee[:, 1]
    uk, idx = jnp.unique(keys, return_inverse=True, size=keys.shape[0], fill_value=-1)
    idx = idx.reshape(-1)
    index = jnp.stack([uk // node_size, uk % node_size], axis=1)
    u = uk.shape[0]

    er = all_matix[:, 3:5].astype(jnp.int32)
    rkeys = er[:, 0] * rel_size + er[:, 1]
    ruk = jnp.unique(rkeys, size=rkeys.shape[0], fill_value=-1)
    rel_index = jnp.stack([ruk // rel_size, ruk % rel_size], axis=1)
    rvals = _row_softmax(rel_index[:, 0], jnp.ones((rel_index.shape[0],), jnp.float32), node_size)
    concept_rel = jax.nn.relu(jax.ops.segment_sum(
        rvals[:, None] * rel_emb[rel_index[:, 1]], rel_index[:, 0], num_segments=node_size))

    ea = attr_matrix[:, 0:2].astype(jnp.int32)
    akeys = ea[:, 0] * attr_size + ea[:, 1]
    auk = jnp.unique(akeys, size=akeys.shape[0], fill_value=-1)
    attr_index = jnp.stack([auk // attr_size, auk % attr_size], axis=1)
    avals = _row_softmax(attr_index[:, 0], jnp.ones((attr_index.shape[0],), jnp.float32), node_size)
    concept_attr = jax.nn.relu(jax.ops.segment_sum(
        avals[:, None] * attr_emb[attr_index[:, 1]], attr_index[:, 0], num_segments=node_size))

    evals = _row_softmax(index[:, 0], jnp.ones((u,), jnp.float32), node_size)
    x = jax.ops.segment_sum(evals[:, None] * ent_emb[index[:, 1]], index[:, 0], num_segments=node_size)

    cnt = jax.ops.segment_sum(jnp.ones((er.shape[0],), jnp.float32), idx, num_segments=u)
    rels_feature = jax.ops.segment_sum(rel_emb[er[:, 1]], idx, num_segments=u) / \
        jnp.where(cnt > 0, cnt, 1.0)[:, None]

    s_cr = concept_rel[index[:, 0]]
    s_ca = concept_attr[index[:, 0]]
    n_cr = concept_rel[index[:, 1]]
    n_ca = concept_attr[index[:, 1]]
    mid = jnp.concatenate([s_cr, s_ca, rels_feature, n_cr, n_ca], axis=-1)

    outputs = []
    for l in range(layers):
        x = jax.nn.relu(x)
        heads = jnp.transpose(x.reshape(node_size, nhead, -1), (1, 0, 2))
        feats = []
        for h in range(nhead):
            e = heads[h]
            feat = jnp.concatenate([e[index[:, 0]], mid, e[index[:, 1]]], axis=-1)
            attn = _attn_scores(feat, attn_kernels[l, h])
            attn = jnp.where(uk >= 0, attn, -jnp.inf)
            attn = jax.nn.softmax(attn, axis=-1)
            attn = _row_softmax(index[:, 0], attn, node_size)
            feats.append(jax.ops.segment_sum(
                e[index[:, 1]] * attn[:, None], index[:, 0], num_segments=node_size))
        x = jnp.tanh(jnp.concatenate(feats, axis=-1))
        outputs.append(x)
    return jnp.concatenate(outputs, axis=-1)


# speedup vs baseline: 1.1340x; 1.1340x over previous
import jax
import jax.numpy as jnp
from jax.experimental import pallas as pl

_BLK = 4000


def _score_block(mid_ref, w_ref, a_ref, b_ref, o_ref):
    s = a_ref[...] + mid_ref[...] @ w_ref[...] + b_ref[...]
    o_ref[...] = jnp.where(s >= 0.0, s, 0.3 * s)


def _attn_scores(mid, wmid, a, b):
    e, k = mid.shape
    h = wmid.shape[1]
    return pl.pallas_call(
        _score_block,
        grid=(e // _BLK,),
        in_specs=[
            pl.BlockSpec((_BLK, k), lambda i: (i, 0)),
            pl.BlockSpec((k, h), lambda i: (0, 0)),
            pl.BlockSpec((_BLK, h), lambda i: (i, 0)),
            pl.BlockSpec((_BLK, h), lambda i: (i, 0)),
        ],
        out_specs=pl.BlockSpec((_BLK, h), lambda i: (i, 0)),
        out_shape=jax.ShapeDtypeStruct((e, h), mid.dtype),
    )(mid, wmid, a, b)


def _row_softmax(rows, vals, nrows):
    m = jax.ops.segment_max(vals, rows, num_segments=nrows)
    ev = jnp.exp(vals - m[rows])
    s = jax.ops.segment_sum(ev, rows, num_segments=nrows)
    return ev / s[rows]


def kernel(ent_emb, rel_emb, attr_emb, all_matix, attr_matrix, attn_kernels):
    node_size = ent_emb.shape[0]
    rel_size = rel_emb.shape[0]
    attr_size = attr_emb.shape[0]
    layers, nhead = attn_kernels.shape[0], attn_kernels.shape[1]

    ee = all_matix[:, 0:2].astype(jnp.int32)
    keys = ee[:, 0] * node_size + ee[:, 1]
    uk, idx = jnp.unique(keys, return_inverse=True, size=keys.shape[0], fill_value=-1)
    idx = idx.reshape(-1)
    index = jnp.stack([uk // node_size, uk % node_size], axis=1)
    u = uk.shape[0]

    er = all_matix[:, 3:5].astype(jnp.int32)
    rkeys = er[:, 0] * rel_size + er[:, 1]
    ruk = jnp.unique(rkeys, size=rkeys.shape[0], fill_value=-1)
    rel_index = jnp.stack([ruk // rel_size, ruk % rel_size], axis=1)
    rvals = _row_softmax(rel_index[:, 0], jnp.ones((rel_index.shape[0],), jnp.float32), node_size)
    concept_rel = jax.nn.relu(jax.ops.segment_sum(
        rvals[:, None] * rel_emb[rel_index[:, 1]], rel_index[:, 0], num_segments=node_size))

    ea = attr_matrix[:, 0:2].astype(jnp.int32)
    akeys = ea[:, 0] * attr_size + ea[:, 1]
    auk = jnp.unique(akeys, size=akeys.shape[0], fill_value=-1)
    attr_index = jnp.stack([auk // attr_size, auk % attr_size], axis=1)
    avals = _row_softmax(attr_index[:, 0], jnp.ones((attr_index.shape[0],), jnp.float32), node_size)
    concept_attr = jax.nn.relu(jax.ops.segment_sum(
        avals[:, None] * attr_emb[attr_index[:, 1]], attr_index[:, 0], num_segments=node_size))

    evals = _row_softmax(index[:, 0], jnp.ones((u,), jnp.float32), node_size)
    x = jax.ops.segment_sum(evals[:, None] * ent_emb[index[:, 1]], index[:, 0], num_segments=node_size)

    cnt = jax.ops.segment_sum(jnp.ones((er.shape[0],), jnp.float32), idx, num_segments=u)
    rels_feature = jax.ops.segment_sum(rel_emb[er[:, 1]], idx, num_segments=u) / \
        jnp.where(cnt > 0, cnt, 1.0)[:, None]

    s_cr = concept_rel[index[:, 0]]
    s_ca = concept_attr[index[:, 0]]
    n_cr = concept_rel[index[:, 1]]
    n_ca = concept_attr[index[:, 1]]
    mid = jnp.concatenate([s_cr, s_ca, rels_feature, n_cr, n_ca], axis=-1)

    dh = ent_emb.shape[1] // nhead
    midw = mid.shape[1]
    outputs = []
    for l in range(layers):
        x = jax.nn.relu(x)
        heads = jnp.transpose(x.reshape(node_size, nhead, -1), (1, 0, 2))
        w = attn_kernels[l][:, :, 0]
        wsrc, wmid, wdst = w[:, :dh], w[:, dh:dh + midw].T, w[:, dh + midw:]
        a_node = jnp.einsum('hnd,hd->nh', heads, wsrc)
        b_node = jnp.einsum('hnd,hd->nh', heads, wdst)
        scores = _attn_scores(mid, wmid, a_node[index[:, 0]], b_node[index[:, 1]])
        feats = []
        for h in range(nhead):
            e = heads[h]
            attn = scores[:, h]
            attn = jnp.where(uk >= 0, attn, -jnp.inf)
            attn = jax.nn.softmax(attn, axis=-1)
            attn = _row_softmax(index[:, 0], attn, node_size)
            feats.append(jax.ops.segment_sum(
                e[index[:, 1]] * attn[:, None], index[:, 0], num_segments=node_size))
        x = jnp.tanh(jnp.concatenate(feats, axis=-1))
        outputs.append(x)
    return jnp.concatenate(outputs, axis=-1)
